# baseline (speedup 1.0000x reference)
import functools

import jax
import jax.numpy as jnp
from jax import lax
from jax.experimental import pallas as pl
from jax.experimental.pallas import tpu as pltpu
from jax.experimental.pallas import tpu_sc as plsc

QUEUE_SIZE = 65536
FEATURE_DIM = 512
BATCH = 16384

_NUM_CORES = 2
_NUM_SUBCORES = 16
_NW = _NUM_CORES * _NUM_SUBCORES
_ROWS_T = BATCH // _NW
_ROWS_Q = (QUEUE_SIZE - BATCH) // _NW

_CHUNK = 120
_NBUF = 2
_LAG = 1


def _build_schedule(regions, sizes):
    sched = []
    k = 0
    for src, dst, base, nrows in regions:
        off = 0
        while off < nrows:
            b = k % len(sizes)
            take = min(sizes[b], nrows - off)
            r = base + off
            sched.append((src.at[pl.ds(r, take)], dst.at[pl.ds(r, take)],
                          b, take))
            off += take
            k += 1
    return sched


def _pipe_copy(sched, bufs, sizes, sem_in, sem_out):
    n = len(sched)
    gat = [None] * len(bufs)
    scat = [None] * len(bufs)
    for i in range(n + _LAG):
        if i < n:
            src, _, b, take = sched[i]
            if scat[b] is not None:
                scat[b].wait()
                scat[b] = None
            dst_buf = bufs[b] if take == sizes[b] else bufs[b].at[pl.ds(0, take)]
            gat[b] = pltpu.async_copy(src, dst_buf, sem_in[b])
        j = i - _LAG
        if j >= 0:
            _, dst, bj, take = sched[j]
            gat[bj].wait()
            src_buf = bufs[bj] if take == sizes[bj] else bufs[bj].at[pl.ds(0, take)]
            scat[bj] = pltpu.async_copy(src_buf, dst, sem_out[bj])
    for s in scat:
        if s is not None:
            s.wait()


def _enqueue_body(tensor, labels, queue, queue_labels, out_q, out_l,
                  buf0, buf1, lbuf_t, lbuf_q,
                  sem_i0, sem_i1, sem_o0, sem_o1, sem_l):
    wid = lax.axis_index("s") * _NUM_CORES + lax.axis_index("c")
    t0 = wid * _ROWS_T
    q0 = BATCH + wid * _ROWS_Q

    lg_t = pltpu.async_copy(labels.at[pl.ds(t0, _ROWS_T)], lbuf_t, sem_l)
    lg_q = pltpu.async_copy(queue_labels.at[pl.ds(q0, _ROWS_Q)], lbuf_q, sem_l)

    regions = (
        (tensor, out_q, t0, _ROWS_T),
        (queue, out_q, q0, _ROWS_Q),
    )
    sched = _build_schedule(regions, (_CHUNK,) * _NBUF)
    _pipe_copy(sched, (buf0, buf1), (_CHUNK,) * _NBUF,
               (sem_i0, sem_i1), (sem_o0, sem_o1))

    lg_t.wait()
    lg_q.wait()
    ls_t = pltpu.async_copy(lbuf_t, out_l.at[pl.ds(t0, _ROWS_T)], sem_l)
    ls_q = pltpu.async_copy(lbuf_q, out_l.at[pl.ds(q0, _ROWS_Q)], sem_l)
    ls_t.wait()
    ls_q.wait()


_enqueue = functools.partial(
    pl.kernel,
    out_type=(
        jax.ShapeDtypeStruct((QUEUE_SIZE, FEATURE_DIM), jnp.float32),
        jax.ShapeDtypeStruct((QUEUE_SIZE,), jnp.int32),
    ),
    mesh=plsc.VectorSubcoreMesh(core_axis_name="c", subcore_axis_name="s"),
    scratch_types=[
        pltpu.VMEM((_CHUNK, FEATURE_DIM), jnp.float32),
        pltpu.VMEM((_CHUNK, FEATURE_DIM), jnp.float32),
        pltpu.VMEM((_ROWS_T,), jnp.int32),
        pltpu.VMEM((_ROWS_Q,), jnp.int32),
        pltpu.SemaphoreType.DMA,
        pltpu.SemaphoreType.DMA,
        pltpu.SemaphoreType.DMA,
        pltpu.SemaphoreType.DMA,
        pltpu.SemaphoreType.DMA,
    ],
)(_enqueue_body)


def kernel(tensor, labels, queue, queue_labels):
    return _enqueue(tensor, labels, queue, queue_labels)
